# Initial kernel scaffold; baseline (speedup 1.0000x reference)
import jax
import jax.numpy as jnp
from jax.experimental import pallas as pl


def kernel(logits):
    raise NotImplementedError("write your pallas kernel here")


# speedup vs baseline: 13.2599x; 13.2599x over previous
import functools

import jax
import jax.numpy as jnp
import numpy as np
from jax import lax
from jax.experimental import pallas as pl
from jax.experimental.pallas import tpu as pltpu
from jax.experimental.pallas import tpu_sc as plsc

ROWS, N = 64, 100000
L = 16
CHUNKS = N // L
UNROLL = 10
NBINS0 = 4096
NBINS1 = 1024
NBINS2 = 1024
NUCLEUS = np.float32(0.9)
IMIN = np.int32(-(2 ** 31))

ROWS_PER_WORKER = 2


def _key_of(v):
    bi = lax.bitcast_convert_type(v, jnp.int32)
    return jnp.where(bi < 0, IMIN - bi, bi)


def _scan_desc(hist_ref, nbins, a0, t):
    iota = lax.iota(jnp.int32, L)
    zero = np.float32(0.0)

    def body(c, carry):
        a, found, bstar, aexcl, gsum, lnb, lnae, lngs = carry
        base = nbins - L * (c + 1)
        chunk = hist_ref[pl.ds(base, L)]
        rev = lax.rev(chunk, (0,))
        cum = plsc.cumsum(rev)
        incl = a + cum
        mask = incl >= t
        lane = jnp.min(jnp.where(mask, iota, L))
        hit = lane < L
        sel = jnp.logical_and(found == 0, hit)
        g_here = jnp.sum(jnp.where(iota == lane, rev, zero))
        i_here = jnp.sum(jnp.where(iota == lane, incl, zero))
        b_here = base + L - 1 - lane
        bstar = jnp.where(sel, b_here, bstar)
        aexcl = jnp.where(sel, i_here - g_here, aexcl)
        gsum = jnp.where(sel, g_here, gsum)
        found = jnp.where(hit, np.int32(1), found)
        lane2 = jnp.max(jnp.where(rev > zero, iota, np.int32(-1)))
        hit2 = lane2 >= 0
        g2 = jnp.sum(jnp.where(iota == lane2, rev, zero))
        i2 = jnp.sum(jnp.where(iota == lane2, incl, zero))
        lnb = jnp.where(hit2, base + L - 1 - lane2, lnb)
        lnae = jnp.where(hit2, i2 - g2, lnae)
        lngs = jnp.where(hit2, g2, lngs)
        a = a + jnp.sum(chunk)
        return a, found, bstar, aexcl, gsum, lnb, lnae, lngs

    init = (a0, np.int32(0), np.int32(0), zero, zero,
            np.int32(0), zero, zero)
    a, found, bstar, aexcl, gsum, lnb, lnae, lngs = lax.fori_loop(
        0, nbins // L, body, init)
    ok = found == 1
    return (jnp.where(ok, bstar, lnb),
            jnp.where(ok, aexcl, lnae),
            jnp.where(ok, gsum, lngs))


def _zero_bins(hist_ref, nbins):
    def body(i, _):
        hist_ref[pl.ds(i * L, L)] = jnp.zeros((L,), jnp.float32)
        return 0

    lax.fori_loop(0, nbins // L, body, 0)


def _do_row(logits_hbm, out_hbm, row_v, hist_v, row):
    pltpu.sync_copy(logits_hbm.at[row], row_v)

    def p1(i, acc):
        base = i * (L * UNROLL)
        for j in range(UNROLL):
            acc = jnp.maximum(acc, row_v[pl.ds(base + j * L, L)])
        return acc

    acc = lax.fori_loop(0, CHUNKS // UNROLL, p1,
                        jnp.full((L,), -jnp.inf, jnp.float32))
    m = jnp.max(acc)

    _zero_bins(hist_v, NBINS0)

    def p2(i, _):
        base = i * (L * UNROLL)
        for j in range(UNROLL):
            v = row_v[pl.ds(base + j * L, L)]
            key = _key_of(v)
            b0 = (key >> 20) + 2048
            e = jnp.exp(v - m)
            plsc.addupdate_scatter(hist_v, [b0], e)
        return 0

    lax.fori_loop(0, CHUNKS // UNROLL, p2, 0)

    def psum(i, s):
        return s + jnp.sum(hist_v[pl.ds(i * L, L)])

    z = lax.fori_loop(0, NBINS0 // L, psum, np.float32(0.0))
    t = NUCLEUS * z

    bs0, a0, _ = _scan_desc(hist_v, NBINS0, np.float32(0.0), t)
    bs0s = bs0 - 2048

    _zero_bins(hist_v, NBINS1)

    def p3(i, _):
        base = i * (L * UNROLL)
        for j in range(UNROLL):
            v = row_v[pl.ds(base + j * L, L)]
            key = _key_of(v)
            msk = (key >> 20) == bs0s
            b1 = (key >> 10) & 1023
            e = jnp.exp(v - m)
            plsc.addupdate_scatter(hist_v, [b1], e, mask=msk)
        return 0

    lax.fori_loop(0, CHUNKS // UNROLL, p3, 0)
    bs1, a1, _ = _scan_desc(hist_v, NBINS1, a0, t)
    prefix21 = (bs0s << 10) | bs1

    _zero_bins(hist_v, NBINS2)

    def p4(i, _):
        base = i * (L * UNROLL)
        for j in range(UNROLL):
            v = row_v[pl.ds(base + j * L, L)]
            key = _key_of(v)
            msk = (key >> 10) == prefix21
            b2 = key & 1023
            e = jnp.exp(v - m)
            plsc.addupdate_scatter(hist_v, [b2], e, mask=msk)
        return 0

    lax.fori_loop(0, CHUNKS // UNROLL, p4, 0)
    bs2, a2, g2 = _scan_desc(hist_v, NBINS2, a1, t)
    kstar = (prefix21 << 10) | bs2
    zk_vec = jnp.zeros((L,), jnp.float32) + (a2 + g2)
    inv_zk = np.float32(1.0) / zk_vec

    zero = np.float32(0.0)

    def p5(i, _):
        base = i * (L * UNROLL)
        for j in range(UNROLL):
            v = row_v[pl.ds(base + j * L, L)]
            key = _key_of(v)
            e = jnp.exp(v - m)
            row_v[pl.ds(base + j * L, L)] = jnp.where(
                key >= kstar, e * inv_zk, zero)
        return 0

    lax.fori_loop(0, CHUNKS // UNROLL, p5, 0)
    pltpu.sync_copy(row_v, out_hbm.at[row])


_MESH = plsc.VectorSubcoreMesh(core_axis_name="c", subcore_axis_name="s")


@functools.partial(
    pl.kernel,
    out_type=jax.ShapeDtypeStruct((ROWS, N), jnp.float32),
    mesh=_MESH,
    compiler_params=pltpu.CompilerParams(needs_layout_passes=False),
    scratch_types=[
        pltpu.VMEM((N,), jnp.float32),
        pltpu.VMEM((NBINS0,), jnp.float32),
    ],
)
def _nucleus_sc(logits_hbm, out_hbm, row_v, hist_v):
    wid = lax.axis_index("s") * 2 + lax.axis_index("c")
    for r in range(ROWS_PER_WORKER):
        _do_row(logits_hbm, out_hbm, row_v, hist_v,
                wid * ROWS_PER_WORKER + r)


def kernel(logits):
    return _nucleus_sc(logits)


# speedup vs baseline: 16.0620x; 1.2113x over previous
import functools

import jax
import jax.numpy as jnp
import numpy as np
from jax import lax
from jax.experimental import pallas as pl
from jax.experimental.pallas import tpu as pltpu
from jax.experimental.pallas import tpu_sc as plsc

ROWS, N = 64, 100000
L = 16
CHUNKS = N // L
UNROLL = 10
NBINS0 = 1024
NBINS1 = 1024
NBINS2 = 1024
NUCLEUS = np.float32(0.9)

ROWS_PER_WORKER = 2


def _scan_desc(hist_ref, nbins, a0, t):
    iota = lax.iota(jnp.int32, L)
    zero = np.float32(0.0)

    def body(c, carry):
        a, found, bstar, aexcl, gsum, lnb, lnae, lngs = carry
        base = nbins - L * (c + 1)
        chunk = hist_ref[pl.ds(base, L)]
        rev = lax.rev(chunk, (0,))
        cum = plsc.cumsum(rev)
        incl = a + cum
        mask = incl >= t
        lane = jnp.min(jnp.where(mask, iota, L))
        hit = lane < L
        sel = jnp.logical_and(found == 0, hit)
        g_here = jnp.sum(jnp.where(iota == lane, rev, zero))
        i_here = jnp.sum(jnp.where(iota == lane, incl, zero))
        b_here = base + L - 1 - lane
        bstar = jnp.where(sel, b_here, bstar)
        aexcl = jnp.where(sel, i_here - g_here, aexcl)
        gsum = jnp.where(sel, g_here, gsum)
        found = jnp.where(hit, np.int32(1), found)
        lane2 = jnp.max(jnp.where(rev > zero, iota, np.int32(-1)))
        hit2 = lane2 >= 0
        g2 = jnp.sum(jnp.where(iota == lane2, rev, zero))
        i2 = jnp.sum(jnp.where(iota == lane2, incl, zero))
        lnb = jnp.where(hit2, base + L - 1 - lane2, lnb)
        lnae = jnp.where(hit2, i2 - g2, lnae)
        lngs = jnp.where(hit2, g2, lngs)
        a = a + jnp.sum(chunk)
        return a, found, bstar, aexcl, gsum, lnb, lnae, lngs

    init = (a0, np.int32(0), np.int32(0), zero, zero,
            np.int32(0), zero, zero)
    a, found, bstar, aexcl, gsum, lnb, lnae, lngs = lax.fori_loop(
        0, nbins // L, body, init)
    ok = found == 1
    return (jnp.where(ok, bstar, lnb),
            jnp.where(ok, aexcl, lnae),
            jnp.where(ok, gsum, lngs))


def _zero_bins(hist_ref, nbins):
    def body(i, _):
        hist_ref[pl.ds(i * L, L)] = jnp.zeros((L,), jnp.float32)
        return 0

    lax.fori_loop(0, nbins // L, body, 0)


def _do_row(logits_hbm, out_hbm, row_v, hist_v, row):
    pltpu.sync_copy(logits_hbm.at[row], row_v)

    def p1(i, acc):
        base = i * (L * UNROLL)
        for j in range(UNROLL):
            acc = jnp.maximum(acc, row_v[pl.ds(base + j * L, L)])
        return acc

    acc = lax.fori_loop(0, CHUNKS // UNROLL, p1,
                        jnp.full((L,), -jnp.inf, jnp.float32))
    m = jnp.max(acc)

    _zero_bins(hist_v, NBINS0)

    def p2(i, _):
        base = i * (L * UNROLL)
        for j in range(UNROLL):
            v = row_v[pl.ds(base + j * L, L)]
            e = jnp.exp(v - m)
            row_v[pl.ds(base + j * L, L)] = e
            b0 = lax.bitcast_convert_type(e, jnp.int32) >> 20
            plsc.addupdate_scatter(hist_v, [b0], e)
        return 0

    lax.fori_loop(0, CHUNKS // UNROLL, p2, 0)

    def psum(i, s):
        return s + jnp.sum(hist_v[pl.ds(i * L, L)])

    z = lax.fori_loop(0, NBINS0 // L, psum, np.float32(0.0))
    t = NUCLEUS * z

    bs0, a0, _ = _scan_desc(hist_v, NBINS0, np.float32(0.0), t)

    _zero_bins(hist_v, NBINS1)

    def p3(i, _):
        base = i * (L * UNROLL)
        for j in range(UNROLL):
            e = row_v[pl.ds(base + j * L, L)]
            bits = lax.bitcast_convert_type(e, jnp.int32)
            msk = (bits >> 20) == bs0
            b1 = (bits >> 10) & 1023
            plsc.addupdate_scatter(hist_v, [b1], e, mask=msk)
        return 0

    lax.fori_loop(0, CHUNKS // UNROLL, p3, 0)
    bs1, a1, _ = _scan_desc(hist_v, NBINS1, a0, t)
    prefix21 = (bs0 << 10) | bs1

    _zero_bins(hist_v, NBINS2)

    def p4(i, _):
        base = i * (L * UNROLL)
        for j in range(UNROLL):
            e = row_v[pl.ds(base + j * L, L)]
            bits = lax.bitcast_convert_type(e, jnp.int32)
            msk = (bits >> 10) == prefix21
            b2 = bits & 1023
            plsc.addupdate_scatter(hist_v, [b2], e, mask=msk)
        return 0

    lax.fori_loop(0, CHUNKS // UNROLL, p4, 0)
    bs2, a2, g2 = _scan_desc(hist_v, NBINS2, a1, t)
    kstar = (prefix21 << 10) | bs2
    zk_vec = jnp.zeros((L,), jnp.float32) + (a2 + g2)
    inv_zk = np.float32(1.0) / zk_vec

    zero = np.float32(0.0)

    def p5(i, _):
        base = i * (L * UNROLL)
        for j in range(UNROLL):
            e = row_v[pl.ds(base + j * L, L)]
            bits = lax.bitcast_convert_type(e, jnp.int32)
            row_v[pl.ds(base + j * L, L)] = jnp.where(
                bits >= kstar, e * inv_zk, zero)
        return 0

    lax.fori_loop(0, CHUNKS // UNROLL, p5, 0)
    pltpu.sync_copy(row_v, out_hbm.at[row])


_MESH = plsc.VectorSubcoreMesh(core_axis_name="c", subcore_axis_name="s")


@functools.partial(
    pl.kernel,
    out_type=jax.ShapeDtypeStruct((ROWS, N), jnp.float32),
    mesh=_MESH,
    compiler_params=pltpu.CompilerParams(needs_layout_passes=False),
    scratch_types=[
        pltpu.VMEM((N,), jnp.float32),
        pltpu.VMEM((NBINS0,), jnp.float32),
    ],
)
def _nucleus_sc(logits_hbm, out_hbm, row_v, hist_v):
    wid = lax.axis_index("s") * 2 + lax.axis_index("c")
    for r in range(ROWS_PER_WORKER):
        _do_row(logits_hbm, out_hbm, row_v, hist_v,
                wid * ROWS_PER_WORKER + r)


def kernel(logits):
    return _nucleus_sc(logits)
